# baseline (speedup 1.0000x reference)
import functools

import jax
import jax.numpy as jnp
from jax import lax
from jax.experimental import pallas as pl
from jax.experimental.pallas import tpu as pltpu

R, SUB, LANE = 128, 8, 4096
NBUF = 4


def _body(inds_ref, x_ref, out_ref, buf_ref, acc_ref, sem_ref):
    acc_ref[...] = jnp.zeros((SUB, LANE), jnp.float32)

    def start(j):
        pltpu.make_async_copy(x_ref.at[j], buf_ref.at[j % NBUF], sem_ref.at[j % NBUF]).start()

    def wait(j):
        pltpu.make_async_copy(x_ref.at[j], buf_ref.at[j % NBUF], sem_ref.at[j % NBUF]).wait()

    for k in range(NBUF - 1):
        @pl.when(inds_ref[k] <= 0)
        def _(k=k):
            start(k)

    def body(j, carry):
        nxt = j + (NBUF - 1)

        @pl.when((nxt < R) & (inds_ref[nxt] <= 0))
        def _():
            start(nxt)

        @pl.when(inds_ref[j] <= 0)
        def _():
            wait(j)
            v = buf_ref[j % NBUF]
            acc_ref[...] = acc_ref[...] + jnp.where(v != v, 1.0, 0.0).astype(jnp.float32)

        return carry

    lax.fori_loop(0, R, body, 0)

    nviol = jnp.sum(acc_ref[...])
    out_ref[...] = jnp.where(nviol == 0.0, 1.0, 0.0).astype(jnp.float32) * jnp.ones(
        (1, 1), jnp.float32
    )


def kernel(x, inds):
    r, c = x.shape
    inds2 = jnp.asarray(inds, dtype=jnp.int32)
    x3 = x.reshape(r, SUB, c // SUB)
    out = pl.pallas_call(
        _body,
        in_specs=[
            pl.BlockSpec(memory_space=pltpu.SMEM),
            pl.BlockSpec(memory_space=pltpu.MemorySpace.HBM),
        ],
        out_specs=pl.BlockSpec(memory_space=pltpu.VMEM),
        out_shape=jax.ShapeDtypeStruct((1, 1), jnp.float32),
        scratch_shapes=[
            pltpu.VMEM((NBUF, SUB, LANE), jnp.float32),
            pltpu.VMEM((SUB, LANE), jnp.float32),
            pltpu.SemaphoreType.DMA((NBUF,)),
        ],
    )(inds2, x3)
    return out.reshape(1)


# speedup vs baseline: 6.1383x; 6.1383x over previous
import jax
import jax.numpy as jnp
from jax.experimental import pallas as pl


def _body(inds_ref, x_ref, out_ref):
    i = pl.program_id(0) + pl.program_id(1)

    @pl.when(i == 0)
    def _init():
        out_ref[...] = jnp.ones((1, 1), jnp.float32)

    xb = x_ref[...]
    m = inds_ref[...] <= 0
    v = jnp.where(m, xb, jnp.float32(0.0))
    ok = jnp.all(v == v)
    out_ref[...] = out_ref[...] * jnp.where(ok, 1.0, 0.0).astype(jnp.float32)


def kernel(x, inds):
    r, c = x.shape
    inds2 = jnp.asarray(inds, dtype=jnp.int32).reshape(r, 1)
    blk_c = 16384
    grid = (c // blk_c, 2)
    out = pl.pallas_call(
        _body,
        grid=grid,
        in_specs=[
            pl.BlockSpec((64, 1), lambda i, j: (j, 0)),
            pl.BlockSpec((64, blk_c), lambda i, j: (j, i)),
        ],
        out_specs=pl.BlockSpec((1, 1), lambda i, j: (0, 0)),
        out_shape=jax.ShapeDtypeStruct((1, 1), jnp.float32),
    )(inds2, x)
    return out.reshape(1)
